# baseline (speedup 1.0000x reference)
import functools

import jax
import jax.numpy as jnp
from jax import lax
from jax.experimental import pallas as pl
from jax.experimental.pallas import tpu as pltpu
from jax.experimental.pallas import tpu_sc as plsc

B = 16384
D_CODE = 128
D_NAME = 128
D_NAT = 32
NAT_BINS = 32
EMB = 256
EPS = 1e-5

_NC = 2
_NS = 16
_NW = _NC * _NS
_NCHUNK = 2
_BH = B // _NCHUNK
_BPW = _BH // _NW
_CHUNK = _BPW // 2


def _sc_body(code_ids, name_ids, code_emb, name_emb,
             cv_out, nv_out, cidx_v, nidx_v, buf0, buf1, buf2,
             sg0, sg1, sg2, sw0, sw1, sw2):
        wid = lax.axis_index("s") * _NC + lax.axis_index("c")
        base = wid * _BPW
        pltpu.sync_copy(code_ids.at[pl.ds(base, _BPW)], cidx_v)
        pltpu.sync_copy(name_ids.at[pl.ds(base, _BPW)], nidx_v)
        g0 = pltpu.async_copy(code_emb.at[cidx_v.at[pl.ds(0, _CHUNK)]], buf0, sg0)
        g1 = pltpu.async_copy(code_emb.at[cidx_v.at[pl.ds(_CHUNK, _CHUNK)]], buf1, sg1)
        g0.wait()
        w0 = pltpu.async_copy(buf0, cv_out.at[pl.ds(base, _CHUNK)], sw0)
        g2 = pltpu.async_copy(name_emb.at[nidx_v.at[pl.ds(0, _CHUNK)]], buf2, sg2)
        g1.wait()
        w1 = pltpu.async_copy(buf1, cv_out.at[pl.ds(base + _CHUNK, _CHUNK)], sw1)
        w0.wait()
        g3 = pltpu.async_copy(name_emb.at[nidx_v.at[pl.ds(_CHUNK, _CHUNK)]], buf0, sg0)
        g2.wait()
        w2 = pltpu.async_copy(buf2, nv_out.at[pl.ds(base, _CHUNK)], sw2)
        g3.wait()
        w3 = pltpu.async_copy(buf0, nv_out.at[pl.ds(base + _CHUNK, _CHUNK)], sw0)
        w1.wait()
        w2.wait()
        w3.wait()


@functools.cache
def _sc_gather():
    return pl.kernel(
        _sc_body,
        mesh=plsc.VectorSubcoreMesh(core_axis_name="c", subcore_axis_name="s"),
        out_type=[
            jax.ShapeDtypeStruct((_BH, D_CODE), jnp.float32),
            jax.ShapeDtypeStruct((_BH, D_NAME), jnp.float32),
        ],
        scratch_types=[
            pltpu.VMEM((_BPW,), jnp.int32),
            pltpu.VMEM((_BPW,), jnp.int32),
            pltpu.VMEM((_CHUNK, D_CODE), jnp.float32),
            pltpu.VMEM((_CHUNK, D_CODE), jnp.float32),
            pltpu.VMEM((_CHUNK, D_CODE), jnp.float32),
            pltpu.SemaphoreType.DMA,
            pltpu.SemaphoreType.DMA,
            pltpu.SemaphoreType.DMA,
            pltpu.SemaphoreType.DMA,
            pltpu.SemaphoreType.DMA,
            pltpu.SemaphoreType.DMA,
        ],
    )


_BM = 4096


def _tc_body_first(cv_ref, nv_ref, nid_ref, nat_ref, w1_ref, w2_ref,
                   w3_ref, b_ref, g_ref, beta_ref, o_ref):
    _tc_compute(cv_ref, nv_ref, nid_ref, nat_ref, w1_ref, w2_ref, w3_ref,
                b_ref, g_ref, beta_ref, o_ref)


def _tc_body_second(_prev_ref, cv_ref, nv_ref, nid_ref, nat_ref, w1_ref,
                    w2_ref, w3_ref, b_ref, g_ref, beta_ref, o_ref):
    _tc_compute(cv_ref, nv_ref, nid_ref, nat_ref, w1_ref, w2_ref, w3_ref,
                b_ref, g_ref, beta_ref, o_ref)


def _tc_compute(cv_ref, nv_ref, nid_ref, nat_ref, w1_ref, w2_ref, w3_ref,
                b_ref, g_ref, beta_ref, o_ref):
    natp = jnp.dot(nat_ref[...], w3_ref[...],
                   preferred_element_type=jnp.float32)
    nids = nid_ref[0, 0, :]
    onehot = (nids[:, None]
              == lax.broadcasted_iota(jnp.int32, (1, NAT_BINS), 1)
              ).astype(jnp.float32)
    x = (jnp.dot(cv_ref[...], w1_ref[...], preferred_element_type=jnp.float32)
         + jnp.dot(nv_ref[...], w2_ref[...], preferred_element_type=jnp.float32)
         + jnp.dot(onehot, natp, preferred_element_type=jnp.float32)
         + b_ref[...])
    mean = jnp.mean(x, axis=-1, keepdims=True)
    xc = x - mean
    var = jnp.mean(xc * xc, axis=-1, keepdims=True)
    o_ref[...] = xc * lax.rsqrt(var + EPS) * g_ref[...] + beta_ref[...]


def _half_specs(blk_off):
    return [
        pl.BlockSpec((_BM, D_CODE), lambda i: (i, 0)),
        pl.BlockSpec((_BM, D_NAME), lambda i: (i, 0)),
        pl.BlockSpec((1, 1, _BM), lambda i: (i + blk_off, 0, 0)),
        pl.BlockSpec((NAT_BINS, D_NAT), lambda i: (0, 0)),
        pl.BlockSpec((D_CODE, EMB), lambda i: (0, 0)),
        pl.BlockSpec((D_NAME, EMB), lambda i: (1, 0)),
        pl.BlockSpec((D_NAT, EMB), lambda i: (8, 0)),
        pl.BlockSpec((1, EMB), lambda i: (0, 0)),
        pl.BlockSpec((1, EMB), lambda i: (0, 0)),
        pl.BlockSpec((1, EMB), lambda i: (0, 0)),
    ]


def _tc_first(cv, nv, nid3, nat, W, b2, g2, beta2):
    return pl.pallas_call(
        _tc_body_first,
        grid=(_BH // _BM,),
        in_specs=_half_specs(0),
        out_specs=pl.BlockSpec((_BM, EMB), lambda i: (i, 0)),
        out_shape=jax.ShapeDtypeStruct((B, EMB), jnp.float32),
    )(cv, nv, nid3, nat, W, W, W, b2, g2, beta2)


def _tc_next(off, prev, cv, nv, nid3, nat, W, b2, g2, beta2):
    return pl.pallas_call(
        _tc_body_second,
        grid=(_BH // _BM,),
        in_specs=[pl.BlockSpec((8, EMB), lambda i: (0, 0))] + _half_specs(off),
        out_specs=pl.BlockSpec((_BM, EMB), lambda i, off=off: (i + off, 0)),
        out_shape=jax.ShapeDtypeStruct((B, EMB), jnp.float32),
        input_output_aliases={0: 0},
    )(prev, cv, nv, nid3, nat, W, W, W, b2, g2, beta2)


def kernel(code_ids, name_ids, nature_ids, code_emb, name_emb, nature_emb,
           W, b, gamma, beta):
    b2 = b.reshape(1, EMB)
    g2 = gamma.reshape(1, EMB)
    beta2 = beta.reshape(1, EMB)
    nid3 = nature_ids.reshape(B // _BM, 1, _BM)

    sc = _sc_gather()
    gathered = [
        sc(code_ids[c * _BH:(c + 1) * _BH], name_ids[c * _BH:(c + 1) * _BH],
           code_emb, name_emb)
        for c in range(_NCHUNK)
    ]
    o = _tc_first(gathered[0][0], gathered[0][1], nid3, nature_emb, W,
                  b2, g2, beta2)
    for c in range(1, _NCHUNK):
        o = _tc_next(c * (_BH // _BM), o, gathered[c][0], gathered[c][1],
                     nid3, nature_emb, W, b2, g2, beta2)
    return o


# speedup vs baseline: 1.0181x; 1.0181x over previous
import functools

import jax
import jax.numpy as jnp
from jax import lax
from jax.experimental import pallas as pl
from jax.experimental.pallas import tpu as pltpu
from jax.experimental.pallas import tpu_sc as plsc

B = 16384
D_CODE = 128
D_NAME = 128
D_NAT = 32
NAT_BINS = 32
EMB = 256
EPS = 1e-5

_NC = 2
_NS = 16
_NW = _NC * _NS
_NCHUNK = 2
_BH = B // _NCHUNK
_BPW = _BH // _NW
_CHUNK = _BPW // 2


def _sc_body(code_ids, name_ids, code_emb, name_emb,
             cv_out, nv_out, cidx_v, nidx_v, buf0, buf1,
             sg0, sg1, sw0, sw1):
        wid = lax.axis_index("s") * _NC + lax.axis_index("c")
        base = wid * _BPW
        pltpu.sync_copy(code_ids.at[pl.ds(base, _BPW)], cidx_v)
        pltpu.sync_copy(name_ids.at[pl.ds(base, _BPW)], nidx_v)
        g0 = pltpu.async_copy(code_emb.at[cidx_v], buf0, sg0)
        g1 = pltpu.async_copy(name_emb.at[nidx_v], buf1, sg1)
        g0.wait()
        w0 = pltpu.async_copy(buf0, cv_out.at[pl.ds(base, _BPW)], sw0)
        g1.wait()
        w1 = pltpu.async_copy(buf1, nv_out.at[pl.ds(base, _BPW)], sw1)
        w0.wait()
        w1.wait()


@functools.cache
def _sc_gather():
    return pl.kernel(
        _sc_body,
        mesh=plsc.VectorSubcoreMesh(core_axis_name="c", subcore_axis_name="s"),
        out_type=[
            jax.ShapeDtypeStruct((_BH, D_CODE), jnp.float32),
            jax.ShapeDtypeStruct((_BH, D_NAME), jnp.float32),
        ],
        scratch_types=[
            pltpu.VMEM((_BPW,), jnp.int32),
            pltpu.VMEM((_BPW,), jnp.int32),
            pltpu.VMEM((_BPW, D_CODE), jnp.float32),
            pltpu.VMEM((_BPW, D_CODE), jnp.float32),
            pltpu.SemaphoreType.DMA,
            pltpu.SemaphoreType.DMA,
            pltpu.SemaphoreType.DMA,
            pltpu.SemaphoreType.DMA,
        ],
    )


_BM = 4096


def _tc_body_first(cv_ref, nv_ref, nid_ref, nat_ref, w1_ref, w2_ref,
                   w3_ref, b_ref, g_ref, beta_ref, o_ref):
    _tc_compute(cv_ref, nv_ref, nid_ref, nat_ref, w1_ref, w2_ref, w3_ref,
                b_ref, g_ref, beta_ref, o_ref)


def _tc_body_second(_prev_ref, cv_ref, nv_ref, nid_ref, nat_ref, w1_ref,
                    w2_ref, w3_ref, b_ref, g_ref, beta_ref, o_ref):
    _tc_compute(cv_ref, nv_ref, nid_ref, nat_ref, w1_ref, w2_ref, w3_ref,
                b_ref, g_ref, beta_ref, o_ref)


def _tc_compute(cv_ref, nv_ref, nid_ref, nat_ref, w1_ref, w2_ref, w3_ref,
                b_ref, g_ref, beta_ref, o_ref):
    natp = jnp.dot(nat_ref[...], w3_ref[...],
                   preferred_element_type=jnp.float32)
    nids = nid_ref[0, 0, :]
    onehot = (nids[:, None]
              == lax.broadcasted_iota(jnp.int32, (1, NAT_BINS), 1)
              ).astype(jnp.float32)
    x = (jnp.dot(cv_ref[...], w1_ref[...], preferred_element_type=jnp.float32)
         + jnp.dot(nv_ref[...], w2_ref[...], preferred_element_type=jnp.float32)
         + jnp.dot(onehot, natp, preferred_element_type=jnp.float32)
         + b_ref[...])
    mean = jnp.mean(x, axis=-1, keepdims=True)
    xc = x - mean
    var = jnp.mean(xc * xc, axis=-1, keepdims=True)
    o_ref[...] = xc * lax.rsqrt(var + EPS) * g_ref[...] + beta_ref[...]


def _half_specs(blk_off):
    return [
        pl.BlockSpec((_BM, D_CODE), lambda i: (i, 0)),
        pl.BlockSpec((_BM, D_NAME), lambda i: (i, 0)),
        pl.BlockSpec((1, 1, _BM), lambda i: (i + blk_off, 0, 0)),
        pl.BlockSpec((NAT_BINS, D_NAT), lambda i: (0, 0)),
        pl.BlockSpec((D_CODE, EMB), lambda i: (0, 0)),
        pl.BlockSpec((D_NAME, EMB), lambda i: (1, 0)),
        pl.BlockSpec((D_NAT, EMB), lambda i: (8, 0)),
        pl.BlockSpec((1, EMB), lambda i: (0, 0)),
        pl.BlockSpec((1, EMB), lambda i: (0, 0)),
        pl.BlockSpec((1, EMB), lambda i: (0, 0)),
    ]


def _tc_first(cv, nv, nid3, nat, W, b2, g2, beta2):
    return pl.pallas_call(
        _tc_body_first,
        grid=(_BH // _BM,),
        in_specs=_half_specs(0),
        out_specs=pl.BlockSpec((_BM, EMB), lambda i: (i, 0)),
        out_shape=jax.ShapeDtypeStruct((B, EMB), jnp.float32),
    )(cv, nv, nid3, nat, W, W, W, b2, g2, beta2)


def _tc_next(off, prev, cv, nv, nid3, nat, W, b2, g2, beta2):
    return pl.pallas_call(
        _tc_body_second,
        grid=(_BH // _BM,),
        in_specs=[pl.BlockSpec((8, EMB), lambda i: (0, 0))] + _half_specs(off),
        out_specs=pl.BlockSpec((_BM, EMB), lambda i, off=off: (i + off, 0)),
        out_shape=jax.ShapeDtypeStruct((B, EMB), jnp.float32),
        input_output_aliases={0: 0},
    )(prev, cv, nv, nid3, nat, W, W, W, b2, g2, beta2)


def kernel(code_ids, name_ids, nature_ids, code_emb, name_emb, nature_emb,
           W, b, gamma, beta):
    b2 = b.reshape(1, EMB)
    g2 = gamma.reshape(1, EMB)
    beta2 = beta.reshape(1, EMB)
    nid3 = nature_ids.reshape(B // _BM, 1, _BM)

    sc = _sc_gather()
    gathered = [
        sc(code_ids[c * _BH:(c + 1) * _BH], name_ids[c * _BH:(c + 1) * _BH],
           code_emb, name_emb)
        for c in range(_NCHUNK)
    ]
    o = _tc_first(gathered[0][0], gathered[0][1], nid3, nature_emb, W,
                  b2, g2, beta2)
    for c in range(1, _NCHUNK):
        o = _tc_next(c * (_BH // _BM), o, gathered[c][0], gathered[c][1],
                     nid3, nature_emb, W, b2, g2, beta2)
    return o


# speedup vs baseline: 1.0494x; 1.0307x over previous
import functools

import jax
import jax.numpy as jnp
from jax import lax
from jax.experimental import pallas as pl
from jax.experimental.pallas import tpu as pltpu
from jax.experimental.pallas import tpu_sc as plsc

B = 16384
D_CODE = 128
D_NAME = 128
D_NAT = 32
NAT_BINS = 32
EMB = 256
EPS = 1e-5

_NC = 2
_NS = 16
_NW = _NC * _NS
_NCHUNK = 2
_BH = B // _NCHUNK
_BPW = _BH // _NW
_CHUNK = _BPW // 2


def _sc_body(code_ids, name_ids, code_emb, name_emb,
             cv_out, nv_out, cidx_v, nidx_v, buf0, buf1,
             sg0, sg1, sw0, sw1, si0, si1):
        wid = lax.axis_index("s") * _NC + lax.axis_index("c")
        base = wid * _BPW
        i0 = pltpu.async_copy(code_ids.at[pl.ds(base, _BPW)], cidx_v, si0)
        i1 = pltpu.async_copy(name_ids.at[pl.ds(base, _BPW)], nidx_v, si1)
        i0.wait()
        g0 = pltpu.async_copy(code_emb.at[cidx_v], buf0, sg0)
        i1.wait()
        g1 = pltpu.async_copy(name_emb.at[nidx_v], buf1, sg1)
        g0.wait()
        w0 = pltpu.async_copy(buf0, cv_out.at[pl.ds(base, _BPW)], sw0)
        g1.wait()
        w1 = pltpu.async_copy(buf1, nv_out.at[pl.ds(base, _BPW)], sw1)
        w0.wait()
        w1.wait()


@functools.cache
def _sc_gather():
    return pl.kernel(
        _sc_body,
        mesh=plsc.VectorSubcoreMesh(core_axis_name="c", subcore_axis_name="s"),
        out_type=[
            jax.ShapeDtypeStruct((_BH, D_CODE), jnp.float32),
            jax.ShapeDtypeStruct((_BH, D_NAME), jnp.float32),
        ],
        scratch_types=[
            pltpu.VMEM((_BPW,), jnp.int32),
            pltpu.VMEM((_BPW,), jnp.int32),
            pltpu.VMEM((_BPW, D_CODE), jnp.float32),
            pltpu.VMEM((_BPW, D_CODE), jnp.float32),
            pltpu.SemaphoreType.DMA,
            pltpu.SemaphoreType.DMA,
            pltpu.SemaphoreType.DMA,
            pltpu.SemaphoreType.DMA,
            pltpu.SemaphoreType.DMA,
            pltpu.SemaphoreType.DMA,
        ],
    )


_BM = 4096


def _tc_body_first(cv_ref, nv_ref, nid_ref, nat_ref, w1_ref, w2_ref,
                   w3_ref, b_ref, g_ref, beta_ref, o_ref):
    _tc_compute(cv_ref, nv_ref, nid_ref, nat_ref, w1_ref, w2_ref, w3_ref,
                b_ref, g_ref, beta_ref, o_ref)


def _tc_body_second(_prev_ref, cv_ref, nv_ref, nid_ref, nat_ref, w1_ref,
                    w2_ref, w3_ref, b_ref, g_ref, beta_ref, o_ref):
    _tc_compute(cv_ref, nv_ref, nid_ref, nat_ref, w1_ref, w2_ref, w3_ref,
                b_ref, g_ref, beta_ref, o_ref)


def _tc_compute(cv_ref, nv_ref, nid_ref, nat_ref, w1_ref, w2_ref, w3_ref,
                b_ref, g_ref, beta_ref, o_ref):
    natp = jnp.dot(nat_ref[...], w3_ref[...],
                   preferred_element_type=jnp.float32)
    nids = nid_ref[0, 0, :]
    onehot = (nids[:, None]
              == lax.broadcasted_iota(jnp.int32, (1, NAT_BINS), 1)
              ).astype(jnp.float32)
    x = (jnp.dot(cv_ref[...], w1_ref[...], preferred_element_type=jnp.float32)
         + jnp.dot(nv_ref[...], w2_ref[...], preferred_element_type=jnp.float32)
         + jnp.dot(onehot, natp, preferred_element_type=jnp.float32)
         + b_ref[...])
    mean = jnp.mean(x, axis=-1, keepdims=True)
    xc = x - mean
    var = jnp.mean(xc * xc, axis=-1, keepdims=True)
    o_ref[...] = xc * lax.rsqrt(var + EPS) * g_ref[...] + beta_ref[...]


def _half_specs(blk_off):
    return [
        pl.BlockSpec((_BM, D_CODE), lambda i: (i, 0)),
        pl.BlockSpec((_BM, D_NAME), lambda i: (i, 0)),
        pl.BlockSpec((1, 1, _BM), lambda i: (i + blk_off, 0, 0)),
        pl.BlockSpec((NAT_BINS, D_NAT), lambda i: (0, 0)),
        pl.BlockSpec((D_CODE, EMB), lambda i: (0, 0)),
        pl.BlockSpec((D_NAME, EMB), lambda i: (1, 0)),
        pl.BlockSpec((D_NAT, EMB), lambda i: (8, 0)),
        pl.BlockSpec((1, EMB), lambda i: (0, 0)),
        pl.BlockSpec((1, EMB), lambda i: (0, 0)),
        pl.BlockSpec((1, EMB), lambda i: (0, 0)),
    ]


def _tc_first(cv, nv, nid3, nat, W, b2, g2, beta2):
    return pl.pallas_call(
        _tc_body_first,
        grid=(_BH // _BM,),
        in_specs=_half_specs(0),
        out_specs=pl.BlockSpec((_BM, EMB), lambda i: (i, 0)),
        out_shape=jax.ShapeDtypeStruct((B, EMB), jnp.float32),
    )(cv, nv, nid3, nat, W, W, W, b2, g2, beta2)


def _tc_next(off, prev, cv, nv, nid3, nat, W, b2, g2, beta2):
    return pl.pallas_call(
        _tc_body_second,
        grid=(_BH // _BM,),
        in_specs=[pl.BlockSpec((8, EMB), lambda i: (0, 0))] + _half_specs(off),
        out_specs=pl.BlockSpec((_BM, EMB), lambda i, off=off: (i + off, 0)),
        out_shape=jax.ShapeDtypeStruct((B, EMB), jnp.float32),
        input_output_aliases={0: 0},
    )(prev, cv, nv, nid3, nat, W, W, W, b2, g2, beta2)


def kernel(code_ids, name_ids, nature_ids, code_emb, name_emb, nature_emb,
           W, b, gamma, beta):
    b2 = b.reshape(1, EMB)
    g2 = gamma.reshape(1, EMB)
    beta2 = beta.reshape(1, EMB)
    nid3 = nature_ids.reshape(B // _BM, 1, _BM)

    sc = _sc_gather()
    gathered = [
        sc(code_ids[c * _BH:(c + 1) * _BH], name_ids[c * _BH:(c + 1) * _BH],
           code_emb, name_emb)
        for c in range(_NCHUNK)
    ]
    o = _tc_first(gathered[0][0], gathered[0][1], nid3, nature_emb, W,
                  b2, g2, beta2)
    for c in range(1, _NCHUNK):
        o = _tc_next(c * (_BH // _BM), o, gathered[c][0], gathered[c][1],
                     nid3, nature_emb, W, b2, g2, beta2)
    return o
